# baseline (speedup 1.0000x reference)
import functools

import jax
import jax.numpy as jnp
from jax import lax
from jax.experimental import pallas as pl
from jax.experimental.pallas import tpu as pltpu
from jax.experimental.pallas import tpu_sc as plsc

N = 10000
D = 128
E = 320000

NC = 2
NS = 16
NW = NC * NS
EPW = E // NW
C = 80
NCHUNK = EPW // C
GROUPS = C // 16
RPS = 624
TAIL = N - NS * RPS

_mesh = plsc.VectorSubcoreMesh(core_axis_name="c", subcore_axis_name="s")


def _gat_body(hs_hbm, hd_hbm, pk_hbm, a_hbm, znd_hbm, zn_hbm,
              num_hbm, den_hbm,
              a_v, pk_0, pk_1, sidx_0, sidx_1, didx_0, didx_1,
              hs_0, hs_1, hd_0, hd_1,
              ex_0, ex_1, num_sh, den_sh, sg0, sg1, sg2, sg3,
              si0, si1, sn0, sn1, se0, se1):
    c = lax.axis_index("c")
    s = lax.axis_index("s")
    wid = c * NS + s
    base = wid * EPW

    pk = (pk_0, pk_1)
    sidx = (sidx_0, sidx_1)
    didx = (didx_0, didx_1)
    hs_b = (hs_0, hs_1)
    hd_b = (hd_0, hd_1)
    ex_b = (ex_0, ex_1)
    sga = (sg0, sg2)
    sgb = (sg1, sg3)
    si = (si0, si1)
    sn = (sn0, sn1)
    se = (se0, se1)

    def idx_start(ci, b):
        off = base + ci * C
        pltpu.async_copy(pk_hbm.at[pl.ds(off, C)], pk[b], si[b])

    def scat_wait(b):
        pltpu.make_async_copy(hd_b[b], num_sh.at[didx[b]], sn[b]).wait()
        pltpu.make_async_copy(ex_b[b], den_sh.at[didx[b]], se[b]).wait()

    def rows_start(ci, b, first=False):
        off = base + ci * C
        pltpu.make_async_copy(pk_hbm.at[pl.ds(off, C)], pk[b], si[b]).wait()
        for g in range(GROUPS):
            w = pk[b][pl.ds(g * 16, 16)]
            sidx[b][pl.ds(g * 16, 16)] = w >> 14
        pltpu.async_copy(hs_hbm.at[sidx[b]], hs_b[b], sga[b])
        if not first:
            @pl.when(ci >= 2)
            def _():
                scat_wait(b)
        for g in range(GROUPS):
            w = pk[b][pl.ds(g * 16, 16)]
            didx[b][pl.ds(g * 16, 16)] = w & 16383
        pltpu.async_copy(hd_hbm.at[didx[b]], hd_b[b], sgb[b])

    def wait(b):
        pltpu.make_async_copy(hs_hbm.at[sidx[b]], hs_b[b],
                              sga[b]).wait()
        pltpu.make_async_copy(hd_hbm.at[didx[b]], hd_b[b],
                              sgb[b]).wait()

    idx_start(0, 0)
    idx_start(1, 1)
    rows_start(0, 0, first=True)

    pltpu.sync_copy(znd_hbm.at[pl.ds(s * RPS, RPS)],
                    num_sh.at[pl.ds(s * RPS, RPS)])

    @pl.when(s == 0)
    def _():
        pltpu.sync_copy(znd_hbm.at[pl.ds(NS * RPS, TAIL)],
                        num_sh.at[pl.ds(NS * RPS, TAIL)])
        pltpu.sync_copy(zn_hbm, den_sh)

    pltpu.sync_copy(a_hbm, a_v)
    plsc.subcore_barrier()

    lanes = lax.iota(jnp.int32, 16)
    av_list = [a_v[pl.ds(kv * 16, 16)] for kv in range(D // 16)]

    def compute(b):
        def gbody(g):
            evec = jnp.zeros((16,), jnp.float32)
            for j in range(16):
                e = g * 16 + j
                acc = jnp.zeros((16,), jnp.float32)
                vs_list = []
                for kv in range(D // 16):
                    vs = hs_b[b][e, pl.ds(kv * 16, 16)]
                    vd = hd_b[b][e, pl.ds(kv * 16, 16)]
                    vs_list.append(vs)
                    t = vs + vd
                    lr = 0.6 * t + 0.4 * jnp.abs(t)
                    acc = acc + av_list[kv] * lr
                parts = [acc[i] for i in range(16)]
                while len(parts) > 1:
                    parts = [parts[i] + parts[i + 1]
                             for i in range(0, len(parts), 2)]
                exs = jnp.exp(jnp.full((16,), parts[0]))
                for kv in range(D // 16):
                    hd_b[b][e, pl.ds(kv * 16, 16)] = vs_list[kv] * exs
                evec = jnp.where(lanes == j, exs, evec)
            ex_v = ex_b[b]
            ex_v[pl.ds(g * 16, 16)] = evec

        plsc.parallel_loop(0, GROUPS)(gbody)
        pltpu.async_copy(hd_b[b], num_sh.at[didx[b]], sn[b], add=True)
        pltpu.async_copy(ex_b[b], den_sh.at[didx[b]], se[b], add=True)

    def chunk_body(i2, carry):
        ci = i2 * 2
        wait(0)
        idx_start(ci + 2, 0)
        rows_start(ci + 1, 1)
        compute(0)
        wait(1)

        @pl.when(ci + 3 < NCHUNK)
        def _():
            idx_start(ci + 3, 1)

        rows_start(ci + 2, 0)
        compute(1)
        return carry

    lax.fori_loop(0, (NCHUNK - 1) // 2, chunk_body, 0, unroll=2)
    wait(0)
    compute(0)
    scat_wait(0)
    scat_wait(1)
    plsc.subcore_barrier()
    pltpu.sync_copy(num_sh.at[pl.ds(s * RPS, RPS)],
                    num_hbm.at[c].at[pl.ds(s * RPS, RPS)])

    @pl.when(s == 0)
    def _():
        pltpu.sync_copy(num_sh.at[pl.ds(NS * RPS, TAIL)],
                        num_hbm.at[c].at[pl.ds(NS * RPS, TAIL)])
        pltpu.sync_copy(den_sh, den_hbm.at[c])


_gat_pass = functools.partial(
    pl.kernel,
    out_type=[jax.ShapeDtypeStruct((NC, N, D), jnp.float32),
              jax.ShapeDtypeStruct((NC, N), jnp.float32)],
    mesh=_mesh,
    scratch_types=[
        pltpu.VMEM((D,), jnp.float32),
        pltpu.VMEM((C,), jnp.int32),
        pltpu.VMEM((C,), jnp.int32),
        pltpu.VMEM((C,), jnp.int32),
        pltpu.VMEM((C,), jnp.int32),
        pltpu.VMEM((C,), jnp.int32),
        pltpu.VMEM((C,), jnp.int32),
        pltpu.VMEM((C, D), jnp.float32),
        pltpu.VMEM((C, D), jnp.float32),
        pltpu.VMEM((C, D), jnp.float32),
        pltpu.VMEM((C, D), jnp.float32),
        pltpu.VMEM((C,), jnp.float32),
        pltpu.VMEM((C,), jnp.float32),
        pltpu.VMEM_SHARED((N, D), jnp.float32),
        pltpu.VMEM_SHARED((N,), jnp.float32),
        pltpu.SemaphoreType.DMA,
        pltpu.SemaphoreType.DMA,
        pltpu.SemaphoreType.DMA,
        pltpu.SemaphoreType.DMA,
        pltpu.SemaphoreType.DMA,
        pltpu.SemaphoreType.DMA,
        pltpu.SemaphoreType.DMA,
        pltpu.SemaphoreType.DMA,
        pltpu.SemaphoreType.DMA,
        pltpu.SemaphoreType.DMA,
    ],
)(_gat_body)


def _sage_body(h_hbm, pk_hbm, znd_hbm, zn_hbm,
               nsum_hbm, deg_hbm,
               pk_0, pk_1, sidx_0, sidx_1, didx_0, didx_1, rows_0, rows_1,
               ones_v, nsum_sh, deg_sh, sg0, sg1, si0, si1, sn0, sn1,
               se0, se1):
    c = lax.axis_index("c")
    s = lax.axis_index("s")
    wid = c * NS + s
    base = wid * EPW
    pk = (pk_0, pk_1)
    sidx = (sidx_0, sidx_1)
    didx = (didx_0, didx_1)
    rows_b = (rows_0, rows_1)
    sg = (sg0, sg1)
    si = (si0, si1)
    sn = (sn0, sn1)
    se = (se0, se1)

    def idx_start(ci, b):
        off = base + ci * C
        pltpu.async_copy(pk_hbm.at[pl.ds(off, C)], pk[b], si[b])

    def scat_wait(b):
        pltpu.make_async_copy(rows_b[b], nsum_sh.at[didx[b]],
                              sn[b]).wait()
        pltpu.make_async_copy(ones_v, deg_sh.at[didx[b]], se[b]).wait()

    def rows_start(ci, b, first=False):
        off = base + ci * C
        pltpu.make_async_copy(pk_hbm.at[pl.ds(off, C)], pk[b],
                              si[b]).wait()
        if not first:
            @pl.when(ci >= 2)
            def _():
                scat_wait(b)
        for g in range(GROUPS):
            w = pk[b][pl.ds(g * 16, 16)]
            sidx[b][pl.ds(g * 16, 16)] = w >> 14
            didx[b][pl.ds(g * 16, 16)] = w & 16383
        pltpu.async_copy(h_hbm.at[sidx[b]], rows_b[b], sg[b])

    def wait(b):
        pltpu.make_async_copy(h_hbm.at[sidx[b]], rows_b[b],
                              sg[b]).wait()

    def scat(b):
        pltpu.async_copy(rows_b[b], nsum_sh.at[didx[b]], sn[b], add=True)
        pltpu.async_copy(ones_v, deg_sh.at[didx[b]], se[b], add=True)

    idx_start(0, 0)
    idx_start(1, 1)
    rows_start(0, 0, first=True)
    pltpu.sync_copy(znd_hbm.at[pl.ds(s * RPS, RPS)],
                    nsum_sh.at[pl.ds(s * RPS, RPS)])

    @pl.when(s == 0)
    def _():
        pltpu.sync_copy(znd_hbm.at[pl.ds(NS * RPS, TAIL)],
                        nsum_sh.at[pl.ds(NS * RPS, TAIL)])
        pltpu.sync_copy(zn_hbm, deg_sh)

    for g in range(GROUPS):
        ones_v[pl.ds(g * 16, 16)] = jnp.ones((16,), jnp.float32)
    plsc.subcore_barrier()

    def chunk_body(i2, carry):
        ci = i2 * 2
        wait(0)
        idx_start(ci + 2, 0)
        rows_start(ci + 1, 1)
        scat(0)
        wait(1)

        @pl.when(ci + 3 < NCHUNK)
        def _():
            idx_start(ci + 3, 1)

        rows_start(ci + 2, 0)
        scat(1)
        return carry

    lax.fori_loop(0, (NCHUNK - 1) // 2, chunk_body, 0)
    wait(0)
    scat(0)
    scat_wait(0)
    scat_wait(1)
    plsc.subcore_barrier()
    pltpu.sync_copy(nsum_sh.at[pl.ds(s * RPS, RPS)],
                    nsum_hbm.at[c].at[pl.ds(s * RPS, RPS)])

    @pl.when(s == 0)
    def _():
        pltpu.sync_copy(nsum_sh.at[pl.ds(NS * RPS, TAIL)],
                        nsum_hbm.at[c].at[pl.ds(NS * RPS, TAIL)])
        pltpu.sync_copy(deg_sh, deg_hbm.at[c])


_sage_pass = functools.partial(
    pl.kernel,
    out_type=[jax.ShapeDtypeStruct((NC, N, D), jnp.float32),
              jax.ShapeDtypeStruct((NC, N), jnp.float32)],
    mesh=_mesh,
    scratch_types=[
        pltpu.VMEM((C,), jnp.int32),
        pltpu.VMEM((C,), jnp.int32),
        pltpu.VMEM((C,), jnp.int32),
        pltpu.VMEM((C,), jnp.int32),
        pltpu.VMEM((C,), jnp.int32),
        pltpu.VMEM((C,), jnp.int32),
        pltpu.VMEM((C, D), jnp.float32),
        pltpu.VMEM((C, D), jnp.float32),
        pltpu.VMEM((C,), jnp.float32),
        pltpu.VMEM_SHARED((N, D), jnp.float32),
        pltpu.VMEM_SHARED((N,), jnp.float32),
        pltpu.SemaphoreType.DMA,
        pltpu.SemaphoreType.DMA,
        pltpu.SemaphoreType.DMA,
        pltpu.SemaphoreType.DMA,
        pltpu.SemaphoreType.DMA,
        pltpu.SemaphoreType.DMA,
        pltpu.SemaphoreType.DMA,
        pltpu.SemaphoreType.DMA,
    ],
)(_sage_body)


BLK = 1000
GRID = N // BLK


def _proj_body(x_ref, wa_ref, wb_ref, hs_ref, hd_ref):
    xb = x_ref[...]
    hs_ref[...] = jnp.dot(xb, wa_ref[...], preferred_element_type=jnp.float32)
    hd_ref[...] = jnp.dot(xb, wb_ref[...], preferred_element_type=jnp.float32)


_proj = pl.pallas_call(
    _proj_body,
    grid=(GRID,),
    in_specs=[
        pl.BlockSpec((BLK, D), lambda i: (i, 0)),
        pl.BlockSpec((D, D), lambda i: (0, 0)),
        pl.BlockSpec((D, D), lambda i: (0, 0)),
    ],
    out_specs=[pl.BlockSpec((BLK, D), lambda i: (i, 0)),
               pl.BlockSpec((BLK, D), lambda i: (i, 0))],
    out_shape=[jax.ShapeDtypeStruct((N, D), jnp.float32),
               jax.ShapeDtypeStruct((N, D), jnp.float32)],
)


def _norm_proj_body(num_ref, den_ref, wa_ref, wb_ref, h_ref, hs_ref, hd_ref):
    i = pl.program_id(0)
    ntot = num_ref[0] + num_ref[1]
    dtot = den_ref[0, pl.ds(i * BLK, BLK), :] + den_ref[1, pl.ds(i * BLK, BLK), :]
    h = ntot / (dtot + 1e-9)
    h_ref[...] = h
    hs_ref[...] = jnp.dot(h, wa_ref[...], preferred_element_type=jnp.float32)
    hd_ref[...] = jnp.dot(h, wb_ref[...], preferred_element_type=jnp.float32)


_norm_proj = pl.pallas_call(
    _norm_proj_body,
    grid=(GRID,),
    in_specs=[
        pl.BlockSpec((NC, BLK, D), lambda i: (0, i, 0)),
        pl.BlockSpec((NC, N, 1), lambda i: (0, 0, 0)),
        pl.BlockSpec((D, D), lambda i: (0, 0)),
        pl.BlockSpec((D, D), lambda i: (0, 0)),
    ],
    out_specs=[pl.BlockSpec((BLK, D), lambda i: (i, 0))] * 3,
    out_shape=[jax.ShapeDtypeStruct((N, D), jnp.float32)] * 3,
)


def _norm_body(num_ref, den_ref, h_ref):
    i = pl.program_id(0)
    ntot = num_ref[0] + num_ref[1]
    dtot = den_ref[0, pl.ds(i * BLK, BLK), :] + den_ref[1, pl.ds(i * BLK, BLK), :]
    h_ref[...] = ntot / (dtot + 1e-9)


_norm = pl.pallas_call(
    _norm_body,
    grid=(GRID,),
    in_specs=[
        pl.BlockSpec((NC, BLK, D), lambda i: (0, i, 0)),
        pl.BlockSpec((NC, N, 1), lambda i: (0, 0, 0)),
    ],
    out_specs=pl.BlockSpec((BLK, D), lambda i: (i, 0)),
    out_shape=jax.ShapeDtypeStruct((N, D), jnp.float32),
)


def _sage_tc_body(nsum_ref, deg_ref, h_ref, wsm_ref, wnm_ref, bm_ref,
                  wsl_ref, wnl_ref, bl_ref, mu_ref, ls_ref):
    i = pl.program_id(0)
    ns = nsum_ref[0] + nsum_ref[1]
    dg = deg_ref[0, pl.ds(i * BLK, BLK), :] + deg_ref[1, pl.ds(i * BLK, BLK), :]
    neigh = ns / jnp.maximum(dg, 1.0)
    h = h_ref[...]
    mu_ref[...] = (jnp.dot(h, wsm_ref[...], preferred_element_type=jnp.float32)
                   + jnp.dot(neigh, wnm_ref[...], preferred_element_type=jnp.float32)
                   + bm_ref[...])
    ls_ref[...] = (jnp.dot(h, wsl_ref[...], preferred_element_type=jnp.float32)
                   + jnp.dot(neigh, wnl_ref[...], preferred_element_type=jnp.float32)
                   + bl_ref[...])


_sage_tc = pl.pallas_call(
    _sage_tc_body,
    grid=(GRID,),
    in_specs=[
        pl.BlockSpec((NC, BLK, D), lambda i: (0, i, 0)),
        pl.BlockSpec((NC, N, 1), lambda i: (0, 0, 0)),
        pl.BlockSpec((BLK, D), lambda i: (i, 0)),
        pl.BlockSpec((D, D), lambda i: (0, 0)),
        pl.BlockSpec((D, D), lambda i: (0, 0)),
        pl.BlockSpec((1, D), lambda i: (0, 0)),
        pl.BlockSpec((D, D), lambda i: (0, 0)),
        pl.BlockSpec((D, D), lambda i: (0, 0)),
        pl.BlockSpec((1, D), lambda i: (0, 0)),
    ],
    out_specs=[pl.BlockSpec((BLK, D), lambda i: (i, 0))] * 2,
    out_shape=[jax.ShapeDtypeStruct((N, D), jnp.float32)] * 2,
)


def kernel(x, adj, W_src1, W_dst1, a1, W_src2, W_dst2, a2, W_src3, W_dst3,
           a3, mu_W_self, mu_W_neigh, mu_b, ls_W_self, ls_W_neigh, ls_b):
    src = adj[0]
    dst = adj[1]
    znd = jnp.zeros((N, D), jnp.float32)
    zn = jnp.zeros((N,), jnp.float32)

    pk = (src << 14) | dst

    hs, hd = _proj(x, W_src1, W_dst1)
    num, den = _gat_pass(hs, hd, pk, a1, znd, zn)
    h, hs, hd = _norm_proj(num, den.reshape(NC, N, 1), W_src2, W_dst2)
    num, den = _gat_pass(hs, hd, pk, a2, znd, zn)
    h, hs, hd = _norm_proj(num, den.reshape(NC, N, 1), W_src3, W_dst3)
    num, den = _gat_pass(hs, hd, pk, a3, znd, zn)
    h3 = _norm(num, den.reshape(NC, N, 1))
    nsum, deg = _sage_pass(h3, pk, znd, zn)
    mu, ls = _sage_tc(nsum, deg.reshape(NC, N, 1), h3,
                      mu_W_self, mu_W_neigh, mu_b.reshape(1, D),
                      ls_W_self, ls_W_neigh, ls_b.reshape(1, D))
    return (mu, ls)


# speedup vs baseline: 1.1604x; 1.1604x over previous
import functools

import jax
import jax.numpy as jnp
from jax import lax
from jax.experimental import pallas as pl
from jax.experimental.pallas import tpu as pltpu
from jax.experimental.pallas import tpu_sc as plsc

N = 10000
D = 128
E = 320000

NC = 2
NS = 16
NW = NC * NS
EPW = E // NW
C = 80
NCHUNK = EPW // C
GROUPS = C // 16
RPS = 624
TAIL = N - NS * RPS

_mesh = plsc.VectorSubcoreMesh(core_axis_name="c", subcore_axis_name="s")


def _gat_body(hs_hbm, hd_hbm, pk_hbm, a_hbm, znd_hbm, zn_hbm,
              num_hbm, den_hbm,
              a_v, pk_0, pk_1, sidx_0, sidx_1, didx_0, didx_1,
              hs_0, hs_1, hd_0, hd_1,
              ex_0, ex_1, num_sh, den_sh, sg0, sg1, sg2, sg3,
              si0, si1, sn0, sn1, se0, se1):
    c = lax.axis_index("c")
    s = lax.axis_index("s")
    wid = c * NS + s
    base = wid * EPW

    pk = (pk_0, pk_1)
    sidx = (sidx_0, sidx_1)
    didx = (didx_0, didx_1)
    hs_b = (hs_0, hs_1)
    hd_b = (hd_0, hd_1)
    ex_b = (ex_0, ex_1)
    sga = (sg0, sg2)
    sgb = (sg1, sg3)
    si = (si0, si1)
    sn = (sn0, sn1)
    se = (se0, se1)

    def idx_start(ci, b):
        off = base + ci * C
        pltpu.async_copy(pk_hbm.at[pl.ds(off, C)], pk[b], si[b])

    def scat_wait(b):
        pltpu.make_async_copy(hd_b[b], num_sh.at[didx[b]], sn[b]).wait()
        pltpu.make_async_copy(ex_b[b], den_sh.at[didx[b]], se[b]).wait()

    def rows_start(ci, b, first=False):
        off = base + ci * C
        pltpu.make_async_copy(pk_hbm.at[pl.ds(off, C)], pk[b], si[b]).wait()
        for g in range(GROUPS):
            w = pk[b][pl.ds(g * 16, 16)]
            sidx[b][pl.ds(g * 16, 16)] = w >> 14
        pltpu.async_copy(hs_hbm.at[sidx[b]], hs_b[b], sga[b])
        if not first:
            @pl.when(ci >= 2)
            def _():
                scat_wait(b)
        for g in range(GROUPS):
            w = pk[b][pl.ds(g * 16, 16)]
            didx[b][pl.ds(g * 16, 16)] = w & 16383
        pltpu.async_copy(hd_hbm.at[didx[b]], hd_b[b], sgb[b])

    def wait(b):
        pltpu.make_async_copy(hs_hbm.at[sidx[b]], hs_b[b],
                              sga[b]).wait()
        pltpu.make_async_copy(hd_hbm.at[didx[b]], hd_b[b],
                              sgb[b]).wait()

    idx_start(0, 0)
    idx_start(1, 1)
    rows_start(0, 0, first=True)

    pltpu.sync_copy(znd_hbm.at[pl.ds(s * RPS, RPS)],
                    num_sh.at[pl.ds(s * RPS, RPS)])

    @pl.when(s == 0)
    def _():
        pltpu.sync_copy(znd_hbm.at[pl.ds(NS * RPS, TAIL)],
                        num_sh.at[pl.ds(NS * RPS, TAIL)])
        pltpu.sync_copy(zn_hbm, den_sh)

    pltpu.sync_copy(a_hbm, a_v)
    plsc.subcore_barrier()

    lanes = lax.iota(jnp.int32, 16)
    av_list = [a_v[pl.ds(kv * 16, 16)] for kv in range(D // 16)]

    def compute(b):
        def gbody(g):
            evec = jnp.zeros((16,), jnp.float32)
            for j in range(16):
                e = g * 16 + j
                acc = jnp.zeros((16,), jnp.float32)
                vs_list = []
                for kv in range(D // 16):
                    vs = hs_b[b][e, pl.ds(kv * 16, 16)]
                    vd = hd_b[b][e, pl.ds(kv * 16, 16)]
                    vs_list.append(vs)
                    t = vs + vd
                    lr = 0.6 * t + 0.4 * jnp.abs(t)
                    acc = acc + av_list[kv] * lr
                parts = [acc[i] for i in range(16)]
                while len(parts) > 1:
                    parts = [parts[i] + parts[i + 1]
                             for i in range(0, len(parts), 2)]
                exs = jnp.exp(jnp.full((16,), parts[0]))
                for kv in range(D // 16):
                    hd_b[b][e, pl.ds(kv * 16, 16)] = vs_list[kv] * exs
                evec = jnp.where(lanes == j, exs, evec)
            ex_v = ex_b[b]
            ex_v[pl.ds(g * 16, 16)] = evec

        plsc.parallel_loop(0, GROUPS)(gbody)
        pltpu.async_copy(hd_b[b], num_sh.at[didx[b]], sn[b], add=True)
        pltpu.async_copy(ex_b[b], den_sh.at[didx[b]], se[b], add=True)

    def chunk_body(i2, carry):
        ci = i2 * 2
        wait(0)
        idx_start(ci + 2, 0)
        rows_start(ci + 1, 1)
        compute(0)
        wait(1)

        @pl.when(ci + 3 < NCHUNK)
        def _():
            idx_start(ci + 3, 1)

        rows_start(ci + 2, 0)
        compute(1)
        return carry

    lax.fori_loop(0, (NCHUNK - 1) // 2, chunk_body, 0)
    wait(0)
    compute(0)
    scat_wait(0)
    scat_wait(1)
    plsc.subcore_barrier()
    pltpu.sync_copy(num_sh.at[pl.ds(s * RPS, RPS)],
                    num_hbm.at[c].at[pl.ds(s * RPS, RPS)])

    @pl.when(s == 0)
    def _():
        pltpu.sync_copy(num_sh.at[pl.ds(NS * RPS, TAIL)],
                        num_hbm.at[c].at[pl.ds(NS * RPS, TAIL)])
        pltpu.sync_copy(den_sh, den_hbm.at[c])


_gat_pass = functools.partial(
    pl.kernel,
    out_type=[jax.ShapeDtypeStruct((NC, N, D), jnp.float32),
              jax.ShapeDtypeStruct((NC, N), jnp.float32)],
    mesh=_mesh,
    scratch_types=[
        pltpu.VMEM((D,), jnp.float32),
        pltpu.VMEM((C,), jnp.int32),
        pltpu.VMEM((C,), jnp.int32),
        pltpu.VMEM((C,), jnp.int32),
        pltpu.VMEM((C,), jnp.int32),
        pltpu.VMEM((C,), jnp.int32),
        pltpu.VMEM((C,), jnp.int32),
        pltpu.VMEM((C, D), jnp.float32),
        pltpu.VMEM((C, D), jnp.float32),
        pltpu.VMEM((C, D), jnp.float32),
        pltpu.VMEM((C, D), jnp.float32),
        pltpu.VMEM((C,), jnp.float32),
        pltpu.VMEM((C,), jnp.float32),
        pltpu.VMEM_SHARED((N, D), jnp.float32),
        pltpu.VMEM_SHARED((N,), jnp.float32),
        pltpu.SemaphoreType.DMA,
        pltpu.SemaphoreType.DMA,
        pltpu.SemaphoreType.DMA,
        pltpu.SemaphoreType.DMA,
        pltpu.SemaphoreType.DMA,
        pltpu.SemaphoreType.DMA,
        pltpu.SemaphoreType.DMA,
        pltpu.SemaphoreType.DMA,
        pltpu.SemaphoreType.DMA,
        pltpu.SemaphoreType.DMA,
    ],
)(_gat_body)


def _sage_body(h_hbm, pk_hbm, znd_hbm, zn_hbm,
               nsum_hbm, deg_hbm,
               pk_0, pk_1, sidx_0, sidx_1, didx_0, didx_1, rows_0, rows_1,
               ones_v, nsum_sh, deg_sh, sg0, sg1, si0, si1, sn0, sn1,
               se0, se1):
    c = lax.axis_index("c")
    s = lax.axis_index("s")
    wid = c * NS + s
    base = wid * EPW
    pk = (pk_0, pk_1)
    sidx = (sidx_0, sidx_1)
    didx = (didx_0, didx_1)
    rows_b = (rows_0, rows_1)
    sg = (sg0, sg1)
    si = (si0, si1)
    sn = (sn0, sn1)
    se = (se0, se1)

    def idx_start(ci, b):
        off = base + ci * C
        pltpu.async_copy(pk_hbm.at[pl.ds(off, C)], pk[b], si[b])

    def scat_wait(b):
        pltpu.make_async_copy(rows_b[b], nsum_sh.at[didx[b]],
                              sn[b]).wait()
        pltpu.make_async_copy(ones_v, deg_sh.at[didx[b]], se[b]).wait()

    def rows_start(ci, b, first=False):
        off = base + ci * C
        pltpu.make_async_copy(pk_hbm.at[pl.ds(off, C)], pk[b],
                              si[b]).wait()
        if not first:
            @pl.when(ci >= 2)
            def _():
                scat_wait(b)
        for g in range(GROUPS):
            w = pk[b][pl.ds(g * 16, 16)]
            sidx[b][pl.ds(g * 16, 16)] = w >> 14
            didx[b][pl.ds(g * 16, 16)] = w & 16383
        pltpu.async_copy(h_hbm.at[sidx[b]], rows_b[b], sg[b])

    def wait(b):
        pltpu.make_async_copy(h_hbm.at[sidx[b]], rows_b[b],
                              sg[b]).wait()

    def scat(b):
        pltpu.async_copy(rows_b[b], nsum_sh.at[didx[b]], sn[b], add=True)
        pltpu.async_copy(ones_v, deg_sh.at[didx[b]], se[b], add=True)

    idx_start(0, 0)
    idx_start(1, 1)
    rows_start(0, 0, first=True)
    pltpu.sync_copy(znd_hbm.at[pl.ds(s * RPS, RPS)],
                    nsum_sh.at[pl.ds(s * RPS, RPS)])

    @pl.when(s == 0)
    def _():
        pltpu.sync_copy(znd_hbm.at[pl.ds(NS * RPS, TAIL)],
                        nsum_sh.at[pl.ds(NS * RPS, TAIL)])
        pltpu.sync_copy(zn_hbm, deg_sh)

    for g in range(GROUPS):
        ones_v[pl.ds(g * 16, 16)] = jnp.ones((16,), jnp.float32)
    plsc.subcore_barrier()

    def chunk_body(i2, carry):
        ci = i2 * 2
        wait(0)
        idx_start(ci + 2, 0)
        rows_start(ci + 1, 1)
        scat(0)
        wait(1)

        @pl.when(ci + 3 < NCHUNK)
        def _():
            idx_start(ci + 3, 1)

        rows_start(ci + 2, 0)
        scat(1)
        return carry

    lax.fori_loop(0, (NCHUNK - 1) // 2, chunk_body, 0)
    wait(0)
    scat(0)
    scat_wait(0)
    scat_wait(1)
    plsc.subcore_barrier()
    pltpu.sync_copy(nsum_sh.at[pl.ds(s * RPS, RPS)],
                    nsum_hbm.at[c].at[pl.ds(s * RPS, RPS)])

    @pl.when(s == 0)
    def _():
        pltpu.sync_copy(nsum_sh.at[pl.ds(NS * RPS, TAIL)],
                        nsum_hbm.at[c].at[pl.ds(NS * RPS, TAIL)])
        pltpu.sync_copy(deg_sh, deg_hbm.at[c])


_sage_pass = functools.partial(
    pl.kernel,
    out_type=[jax.ShapeDtypeStruct((NC, N, D), jnp.float32),
              jax.ShapeDtypeStruct((NC, N), jnp.float32)],
    mesh=_mesh,
    scratch_types=[
        pltpu.VMEM((C,), jnp.int32),
        pltpu.VMEM((C,), jnp.int32),
        pltpu.VMEM((C,), jnp.int32),
        pltpu.VMEM((C,), jnp.int32),
        pltpu.VMEM((C,), jnp.int32),
        pltpu.VMEM((C,), jnp.int32),
        pltpu.VMEM((C, D), jnp.float32),
        pltpu.VMEM((C, D), jnp.float32),
        pltpu.VMEM((C,), jnp.float32),
        pltpu.VMEM_SHARED((N, D), jnp.float32),
        pltpu.VMEM_SHARED((N,), jnp.float32),
        pltpu.SemaphoreType.DMA,
        pltpu.SemaphoreType.DMA,
        pltpu.SemaphoreType.DMA,
        pltpu.SemaphoreType.DMA,
        pltpu.SemaphoreType.DMA,
        pltpu.SemaphoreType.DMA,
        pltpu.SemaphoreType.DMA,
        pltpu.SemaphoreType.DMA,
    ],
)(_sage_body)


BLK = 1000
GRID = N // BLK


def _proj_body(x_ref, wa_ref, wb_ref, hs_ref, hd_ref):
    xb = x_ref[...]
    hs_ref[...] = jnp.dot(xb, wa_ref[...], preferred_element_type=jnp.float32)
    hd_ref[...] = jnp.dot(xb, wb_ref[...], preferred_element_type=jnp.float32)


_proj = pl.pallas_call(
    _proj_body,
    grid=(GRID,),
    in_specs=[
        pl.BlockSpec((BLK, D), lambda i: (i, 0)),
        pl.BlockSpec((D, D), lambda i: (0, 0)),
        pl.BlockSpec((D, D), lambda i: (0, 0)),
    ],
    out_specs=[pl.BlockSpec((BLK, D), lambda i: (i, 0)),
               pl.BlockSpec((BLK, D), lambda i: (i, 0))],
    out_shape=[jax.ShapeDtypeStruct((N, D), jnp.float32),
               jax.ShapeDtypeStruct((N, D), jnp.float32)],
)


def _norm_proj_body(num_ref, den_ref, wa_ref, wb_ref, h_ref, hs_ref, hd_ref):
    i = pl.program_id(0)
    ntot = num_ref[0] + num_ref[1]
    dtot = den_ref[0, pl.ds(i * BLK, BLK), :] + den_ref[1, pl.ds(i * BLK, BLK), :]
    h = ntot / (dtot + 1e-9)
    h_ref[...] = h
    hs_ref[...] = jnp.dot(h, wa_ref[...], preferred_element_type=jnp.float32)
    hd_ref[...] = jnp.dot(h, wb_ref[...], preferred_element_type=jnp.float32)


_norm_proj = pl.pallas_call(
    _norm_proj_body,
    grid=(GRID,),
    in_specs=[
        pl.BlockSpec((NC, BLK, D), lambda i: (0, i, 0)),
        pl.BlockSpec((NC, N, 1), lambda i: (0, 0, 0)),
        pl.BlockSpec((D, D), lambda i: (0, 0)),
        pl.BlockSpec((D, D), lambda i: (0, 0)),
    ],
    out_specs=[pl.BlockSpec((BLK, D), lambda i: (i, 0))] * 3,
    out_shape=[jax.ShapeDtypeStruct((N, D), jnp.float32)] * 3,
)


def _norm_body(num_ref, den_ref, h_ref):
    i = pl.program_id(0)
    ntot = num_ref[0] + num_ref[1]
    dtot = den_ref[0, pl.ds(i * BLK, BLK), :] + den_ref[1, pl.ds(i * BLK, BLK), :]
    h_ref[...] = ntot / (dtot + 1e-9)


_norm = pl.pallas_call(
    _norm_body,
    grid=(GRID,),
    in_specs=[
        pl.BlockSpec((NC, BLK, D), lambda i: (0, i, 0)),
        pl.BlockSpec((NC, N, 1), lambda i: (0, 0, 0)),
    ],
    out_specs=pl.BlockSpec((BLK, D), lambda i: (i, 0)),
    out_shape=jax.ShapeDtypeStruct((N, D), jnp.float32),
)


def _sage_tc_body(nsum_ref, deg_ref, h_ref, wsm_ref, wnm_ref, bm_ref,
                  wsl_ref, wnl_ref, bl_ref, mu_ref, ls_ref):
    i = pl.program_id(0)
    ns = nsum_ref[0] + nsum_ref[1]
    dg = deg_ref[0, pl.ds(i * BLK, BLK), :] + deg_ref[1, pl.ds(i * BLK, BLK), :]
    neigh = ns / jnp.maximum(dg, 1.0)
    h = h_ref[...]
    mu_ref[...] = (jnp.dot(h, wsm_ref[...], preferred_element_type=jnp.float32)
                   + jnp.dot(neigh, wnm_ref[...], preferred_element_type=jnp.float32)
                   + bm_ref[...])
    ls_ref[...] = (jnp.dot(h, wsl_ref[...], preferred_element_type=jnp.float32)
                   + jnp.dot(neigh, wnl_ref[...], preferred_element_type=jnp.float32)
                   + bl_ref[...])


_sage_tc = pl.pallas_call(
    _sage_tc_body,
    grid=(GRID,),
    in_specs=[
        pl.BlockSpec((NC, BLK, D), lambda i: (0, i, 0)),
        pl.BlockSpec((NC, N, 1), lambda i: (0, 0, 0)),
        pl.BlockSpec((BLK, D), lambda i: (i, 0)),
        pl.BlockSpec((D, D), lambda i: (0, 0)),
        pl.BlockSpec((D, D), lambda i: (0, 0)),
        pl.BlockSpec((1, D), lambda i: (0, 0)),
        pl.BlockSpec((D, D), lambda i: (0, 0)),
        pl.BlockSpec((D, D), lambda i: (0, 0)),
        pl.BlockSpec((1, D), lambda i: (0, 0)),
    ],
    out_specs=[pl.BlockSpec((BLK, D), lambda i: (i, 0))] * 2,
    out_shape=[jax.ShapeDtypeStruct((N, D), jnp.float32)] * 2,
)


def kernel(x, adj, W_src1, W_dst1, a1, W_src2, W_dst2, a2, W_src3, W_dst3,
           a3, mu_W_self, mu_W_neigh, mu_b, ls_W_self, ls_W_neigh, ls_b):
    src = adj[0]
    dst = adj[1]
    znd = jnp.zeros((N, D), jnp.float32)
    zn = jnp.zeros((N,), jnp.float32)

    pk = (src << 14) | dst

    hs, hd = _proj(x, W_src1, W_dst1)
    num, den = _gat_pass(hs, hd, pk, a1, znd, zn)
    h, hs, hd = _norm_proj(num, den.reshape(NC, N, 1), W_src2, W_dst2)
    num, den = _gat_pass(hs, hd, pk, a2, znd, zn)
    h, hs, hd = _norm_proj(num, den.reshape(NC, N, 1), W_src3, W_dst3)
    num, den = _gat_pass(hs, hd, pk, a3, znd, zn)
    h3 = _norm(num, den.reshape(NC, N, 1))
    nsum, deg = _sage_pass(h3, pk, znd, zn)
    mu, ls = _sage_tc(nsum, deg.reshape(NC, N, 1), h3,
                      mu_W_self, mu_W_neigh, mu_b.reshape(1, D),
                      ls_W_self, ls_W_neigh, ls_b.reshape(1, D))
    return (mu, ls)
